# baseline (speedup 1.0000x reference)
import functools

import jax
import jax.numpy as jnp
from jax import lax
from jax.experimental import pallas as pl
from jax.experimental.pallas import tpu as pltpu
from jax.experimental.pallas import tpu_sc as plsc

_D = 64
_SCALE = 8.0
_LANES = 128


def _sc_transpose(embT):
    nd, V = embT.shape
    info = plsc.get_sparse_core_info()
    NW = info.num_cores * info.num_subcores
    nfull = V // _LANES
    mesh = plsc.VectorSubcoreMesh(core_axis_name="c", subcore_axis_name="s")

    @functools.partial(
        pl.kernel,
        mesh=mesh,
        compiler_params=pltpu.CompilerParams(
            use_tc_tiling_on_sc=True, needs_layout_passes=False
        ),
        out_type=jax.ShapeDtypeStruct((V // 2, 2 * nd), jnp.float32),
        scratch_types=[
            pltpu.VMEM((3, _D, _LANES), jnp.float32),
            pltpu.VMEM((2, _D, _LANES), jnp.float32),
            pltpu.SemaphoreType.DMA,
            pltpu.SemaphoreType.DMA,
        ],
    )
    def k(embT_hbm, out_hbm, in_v, o_v, isem, osem):
        wid = lax.axis_index("s") * info.num_cores + lax.axis_index("c")
        nb = 244 + jnp.where(wid < nfull - 32 * 244, 1, 0)
        iota16 = lax.iota(jnp.int32, 16)

        def blkid(b):
            return wid + NW * b

        for pb in (0, 1):
            pltpu.async_copy(
                embT_hbm.at[:, pl.ds(blkid(pb) * _LANES, _LANES)],
                in_v.at[pb], isem)

        def transpose_block(obuf, blk, rows, ibuf):
            def kbody(kk, c2):
                qvec = (iota16 + kk) & (rows - 1)
                vals = []
                for g in range(_LANES // 16):
                    rvec = (iota16 + (g % 4) * 16) & (_D - 1)
                    cvec = 2 * qvec + (g // 4)
                    vals.append(
                        plsc.load_gather(in_v.at[ibuf], [rvec, cvec]))
                for g in range(_LANES // 16):
                    plsc.store_scatter(
                        o_v.at[obuf], [qvec, iota16 + g * 16], vals[g])
                return c2

            lax.fori_loop(0, rows, kbody, 0, unroll=8)
            pltpu.async_copy(
                o_v.at[obuf].at[pl.ds(0, rows)],
                out_hbm.at[pl.ds(blk * (_LANES // 2), rows)], osem)

        def bbody(b, c):
            buf = lax.rem(b, 3)
            blk = blkid(b)
            pltpu.make_async_copy(
                embT_hbm.at[:, pl.ds(blk * _LANES, _LANES)],
                in_v.at[buf], isem).wait()

            @pl.when(b + 2 < nb)
            def _():
                pltpu.async_copy(
                    embT_hbm.at[:, pl.ds(blkid(b + 2) * _LANES, _LANES)],
                    in_v.at[lax.rem(b + 2, 3)], isem)

            @pl.when(b >= 2)
            def _():
                pltpu.make_async_copy(
                    o_v.at[lax.rem(b, 2)],
                    out_hbm.at[pl.ds(0, _D)], osem).wait()

            transpose_block(lax.rem(b, 2), blk, _D, buf)
            return c

        lax.fori_loop(0, nb, bbody, 0)

        def drain(t):
            @pl.when(t < nb)
            def _():
                pltpu.make_async_copy(
                    o_v.at[t % 2], out_hbm.at[pl.ds(0, _D)], osem).wait()

        drain(nb - 2)
        drain(nb - 1)

    return k(embT)


def _sc_embed(xT, tab2):
    nr, ntok = xT.shape
    info = plsc.get_sparse_core_info()

    mesh = plsc.VectorSubcoreMesh(core_axis_name="c", subcore_axis_name="s")

    @functools.partial(
        pl.kernel,
        mesh=mesh,
        compiler_params=pltpu.CompilerParams(
            use_tc_tiling_on_sc=True, needs_layout_passes=False
        ),
        out_type=jax.ShapeDtypeStruct((nr, _D, ntok), jnp.float32),
        scratch_types=[
            pltpu.VMEM((nr, _LANES), jnp.int32),
            pltpu.VMEM((nr, _LANES), jnp.int32),
            pltpu.VMEM((3, _LANES, _LANES), jnp.float32),
            pltpu.VMEM((2, _D, _LANES), jnp.float32),
            pltpu.SemaphoreType.DMA,
            pltpu.SemaphoreType.DMA,
        ],
    )
    def k(xT_hbm, tab_hbm, out_hbm, idx_v, hidx_v, rows_v, out_v, gsem, osem):
        wid = lax.axis_index("s") * info.num_cores + lax.axis_index("c")
        a0 = wid * _LANES
        pltpu.sync_copy(xT_hbm.at[:, pl.ds(a0, _LANES)], idx_v)

        def hbody(i, c):
            for g in range(_LANES // 16):
                sl = pl.ds(g * 16, 16)
                hidx_v[i, sl] = lax.shift_right_logical(idx_v[i, sl], 1)
            return c

        lax.fori_loop(0, nr, hbody, 0, unroll=4)

        iota16 = lax.iota(jnp.int32, 16)
        for pr in (0, 1):
            pltpu.async_copy(tab_hbm.at[hidx_v.at[pr]], rows_v.at[pr], gsem)

        def rbody(r, c):
            buf = lax.rem(r, 3)
            obuf = lax.rem(r, 2)
            pltpu.make_async_copy(
                tab_hbm.at[hidx_v.at[r]], rows_v.at[buf], gsem).wait()

            @pl.when(r + 2 < nr)
            def _():
                pltpu.async_copy(
                    tab_hbm.at[hidx_v.at[r + 2]],
                    rows_v.at[lax.rem(r + 2, 3)], gsem)

            @pl.when(r >= 2)
            def _():
                pltpu.make_async_copy(
                    out_v.at[obuf],
                    out_hbm.at[r - 2, :, pl.ds(a0, _LANES)], osem).wait()

            doffs = []
            for g in range(_LANES // 16):
                sl = pl.ds(g * 16, 16)
                doffs.append((idx_v[r, sl] & 1) * _D)

            def kbody(kk, c2):
                dvec = (iota16 + kk) & (_D - 1)
                vals = [
                    plsc.load_gather(
                        rows_v.at[buf], [iota16 + (g * 16), doffs[g] + dvec])
                    for g in range(_LANES // 16)
                ]
                for g in range(_LANES // 16):
                    plsc.store_scatter(
                        out_v.at[obuf], [dvec, iota16 + (g * 16)],
                        vals[g] * _SCALE)
                return c2

            lax.fori_loop(0, _D, kbody, 0, unroll=8)
            pltpu.async_copy(
                out_v.at[obuf], out_hbm.at[r, :, pl.ds(a0, _LANES)], osem)
            return c

        lax.fori_loop(0, nr, rbody, 0)
        for t in (nr - 2, nr - 1):
            pltpu.make_async_copy(
                out_v.at[t % 2], out_hbm.at[t, :, pl.ds(a0, _LANES)],
                osem).wait()

    return k(xT, tab2)


def kernel(x, emb_weight):
    b, s = x.shape
    v, d = emb_weight.shape
    xT = x.T.astype(jnp.int32)
    tab2 = _sc_transpose(emb_weight.T)
    ntail = v % _LANES
    if ntail:
        tail = emb_weight[v - ntail:].reshape(ntail // 2, 2 * d)
        tab2 = lax.dynamic_update_slice(
            tab2, tail, ((v - ntail) // 2, 0))
    outT = _sc_embed(xT, tab2)
    return outT.transpose(2, 0, 1)


# speedup vs baseline: 1.0052x; 1.0052x over previous
import functools

import jax
import jax.numpy as jnp
from jax import lax
from jax.experimental import pallas as pl
from jax.experimental.pallas import tpu as pltpu
from jax.experimental.pallas import tpu_sc as plsc

_D = 64
_SCALE = 8.0
_LANES = 128


def _sc_transpose(embT):
    nd, V = embT.shape
    info = plsc.get_sparse_core_info()
    NW = info.num_cores * info.num_subcores
    nfull = V // _LANES
    mesh = plsc.VectorSubcoreMesh(core_axis_name="c", subcore_axis_name="s")

    @functools.partial(
        pl.kernel,
        mesh=mesh,
        compiler_params=pltpu.CompilerParams(
            use_tc_tiling_on_sc=True, needs_layout_passes=False
        ),
        out_type=jax.ShapeDtypeStruct((V // 2, 2 * nd), jnp.float32),
        scratch_types=[
            pltpu.VMEM((3, _D, _LANES), jnp.float32),
            pltpu.VMEM((2, _D, _LANES), jnp.float32),
            pltpu.SemaphoreType.DMA,
            pltpu.SemaphoreType.DMA,
        ],
    )
    def k(embT_hbm, out_hbm, in_v, o_v, isem, osem):
        wid = lax.axis_index("s") * info.num_cores + lax.axis_index("c")
        nb = 244 + jnp.where(wid < nfull - 32 * 244, 1, 0)
        iota16 = lax.iota(jnp.int32, 16)

        def blkid(b):
            return wid + NW * b

        for pb in (0, 1):
            pltpu.async_copy(
                embT_hbm.at[:, pl.ds(blkid(pb) * _LANES, _LANES)],
                in_v.at[pb], isem)

        def transpose_block(obuf, blk, rows, ibuf):
            def kbody(kk, c2):
                qvec = (iota16 + kk) & (rows - 1)
                vals = []
                for g in range(_LANES // 16):
                    rvec = (iota16 + (g % 4) * 16) & (_D - 1)
                    cvec = 2 * qvec + (g // 4)
                    vals.append(
                        plsc.load_gather(in_v.at[ibuf], [rvec, cvec]))
                for g in range(_LANES // 16):
                    plsc.store_scatter(
                        o_v.at[obuf], [qvec, iota16 + g * 16], vals[g])
                return c2

            lax.fori_loop(0, rows, kbody, 0, unroll=4)
            pltpu.async_copy(
                o_v.at[obuf].at[pl.ds(0, rows)],
                out_hbm.at[pl.ds(blk * (_LANES // 2), rows)], osem)

        def bbody(b, c):
            buf = lax.rem(b, 3)
            blk = blkid(b)
            pltpu.make_async_copy(
                embT_hbm.at[:, pl.ds(blk * _LANES, _LANES)],
                in_v.at[buf], isem).wait()

            @pl.when(b + 2 < nb)
            def _():
                pltpu.async_copy(
                    embT_hbm.at[:, pl.ds(blkid(b + 2) * _LANES, _LANES)],
                    in_v.at[lax.rem(b + 2, 3)], isem)

            @pl.when(b >= 2)
            def _():
                pltpu.make_async_copy(
                    o_v.at[lax.rem(b, 2)],
                    out_hbm.at[pl.ds(0, _D)], osem).wait()

            transpose_block(lax.rem(b, 2), blk, _D, buf)
            return c

        lax.fori_loop(0, nb, bbody, 0)

        def drain(t):
            @pl.when(t < nb)
            def _():
                pltpu.make_async_copy(
                    o_v.at[t % 2], out_hbm.at[pl.ds(0, _D)], osem).wait()

        drain(nb - 2)
        drain(nb - 1)

    return k(embT)


def _sc_embed(xT, tab2):
    nr, ntok = xT.shape
    info = plsc.get_sparse_core_info()

    mesh = plsc.VectorSubcoreMesh(core_axis_name="c", subcore_axis_name="s")

    @functools.partial(
        pl.kernel,
        mesh=mesh,
        compiler_params=pltpu.CompilerParams(
            use_tc_tiling_on_sc=True, needs_layout_passes=False
        ),
        out_type=jax.ShapeDtypeStruct((nr, _D, ntok), jnp.float32),
        scratch_types=[
            pltpu.VMEM((nr, _LANES), jnp.int32),
            pltpu.VMEM((nr, _LANES), jnp.int32),
            pltpu.VMEM((3, _LANES, _LANES), jnp.float32),
            pltpu.VMEM((2, _D, _LANES), jnp.float32),
            pltpu.SemaphoreType.DMA,
            pltpu.SemaphoreType.DMA,
        ],
    )
    def k(xT_hbm, tab_hbm, out_hbm, idx_v, hidx_v, rows_v, out_v, gsem, osem):
        wid = lax.axis_index("s") * info.num_cores + lax.axis_index("c")
        a0 = wid * _LANES
        pltpu.sync_copy(xT_hbm.at[:, pl.ds(a0, _LANES)], idx_v)

        def hbody(i, c):
            for g in range(_LANES // 16):
                sl = pl.ds(g * 16, 16)
                hidx_v[i, sl] = lax.shift_right_logical(idx_v[i, sl], 1)
            return c

        lax.fori_loop(0, nr, hbody, 0, unroll=4)

        iota16 = lax.iota(jnp.int32, 16)
        for pr in (0, 1):
            pltpu.async_copy(tab_hbm.at[hidx_v.at[pr]], rows_v.at[pr], gsem)

        def rbody(r, c):
            buf = lax.rem(r, 3)
            obuf = lax.rem(r, 2)
            pltpu.make_async_copy(
                tab_hbm.at[hidx_v.at[r]], rows_v.at[buf], gsem).wait()

            @pl.when(r + 2 < nr)
            def _():
                pltpu.async_copy(
                    tab_hbm.at[hidx_v.at[r + 2]],
                    rows_v.at[lax.rem(r + 2, 3)], gsem)

            @pl.when(r >= 2)
            def _():
                pltpu.make_async_copy(
                    out_v.at[obuf],
                    out_hbm.at[r - 2, :, pl.ds(a0, _LANES)], osem).wait()

            doffs = []
            for g in range(_LANES // 16):
                sl = pl.ds(g * 16, 16)
                doffs.append((idx_v[r, sl] & 1) * _D)

            def kbody(kk, c2):
                dvec = (iota16 + kk) & (_D - 1)
                vals = [
                    plsc.load_gather(
                        rows_v.at[buf], [iota16 + (g * 16), doffs[g] + dvec])
                    for g in range(_LANES // 16)
                ]
                for g in range(_LANES // 16):
                    plsc.store_scatter(
                        out_v.at[obuf], [dvec, iota16 + (g * 16)],
                        vals[g] * _SCALE)
                return c2

            lax.fori_loop(0, _D, kbody, 0, unroll=4)
            pltpu.async_copy(
                out_v.at[obuf], out_hbm.at[r, :, pl.ds(a0, _LANES)], osem)
            return c

        lax.fori_loop(0, nr, rbody, 0)
        for t in (nr - 2, nr - 1):
            pltpu.make_async_copy(
                out_v.at[t % 2], out_hbm.at[t, :, pl.ds(a0, _LANES)],
                osem).wait()

    return k(xT, tab2)


def kernel(x, emb_weight):
    b, s = x.shape
    v, d = emb_weight.shape
    xT = x.T.astype(jnp.int32)
    tab2 = _sc_transpose(emb_weight.T)
    ntail = v % _LANES
    if ntail:
        tail = emb_weight[v - ntail:].reshape(ntail // 2, 2 * d)
        tab2 = lax.dynamic_update_slice(
            tab2, tail, ((v - ntail) // 2, 0))
    outT = _sc_embed(xT, tab2)
    return outT.transpose(2, 0, 1)
